# baseline (speedup 1.0000x reference)
import functools

import jax
import jax.numpy as jnp
import numpy as np
from jax import lax
from jax.experimental import pallas as pl
from jax.experimental.pallas import tpu as pltpu
from jax.experimental.pallas import tpu_sc as plsc

NUM_CLASSES = 1000
EMBED_DIM = 512
INTER_DIM = 2048
BATCH = 16384
HALF_DIM = EMBED_DIM // 2
N_GRID = 128


_FREQ = np.exp(
    -np.log(10000.0) * np.arange(HALF_DIM, dtype=np.float32) / HALF_DIM
).astype(np.float32).reshape(1, HALF_DIM)


def _grid_mlp_body(freq_ref, w1_ref, b1_ref, w2_ref, b2_ref, t_ref):
    g = lax.broadcasted_iota(jnp.int32, (N_GRID, HALF_DIM), 0).astype(jnp.float32)
    x = (g * (1.0 / (N_GRID - 1))) * freq_ref[0, :][None, :]
    y = x * x
    s = x * (1.0 + y * (-1.0 / 6.0 + y * (1.0 / 120.0 + y * (-1.0 / 5040.0))))
    c = 1.0 + y * (-0.5 + y * (1.0 / 24.0 + y * (-1.0 / 720.0)))
    emb = jnp.concatenate([s, c], axis=-1).astype(jnp.bfloat16)
    h = lax.dot_general(
        emb, w1_ref[...].astype(jnp.bfloat16),
        (((1,), (1,)), ((), ())),
        preferred_element_type=jnp.float32,
    ) + b1_ref[0, :][None, :]
    h = h * jax.nn.sigmoid(h)
    h = h.astype(jnp.bfloat16)
    out = lax.dot_general(
        h, w2_ref[...].astype(jnp.bfloat16),
        (((1,), (1,)), ((), ())),
        preferred_element_type=jnp.float32,
    ) + b2_ref[0, :][None, :]
    t_ref[...] = out


def _tc_grid_mlp(freq, w1, b1, w2, b2):
    return pl.pallas_call(
        _grid_mlp_body,
        out_shape=jax.ShapeDtypeStruct((N_GRID, EMBED_DIM), jnp.float32),
    )(freq, w1, b1, w2, b2)


_NC = 2
_NS = 16
_NW = _NC * _NS
_B_PER_W = BATCH // _NW
_CHUNK = 32
_N_CHUNKS = _B_PER_W // _CHUNK


@functools.cache
def _make_sc_fused():
    mesh = plsc.VectorSubcoreMesh(core_axis_name="c", subcore_axis_name="s")

    @functools.partial(
        pl.kernel,
        mesh=mesh,
        out_type=jax.ShapeDtypeStruct((BATCH, EMBED_DIM), jnp.float32),
        scratch_types=[
            pltpu.VMEM((_B_PER_W,), jnp.int32),
            pltpu.VMEM((_B_PER_W,), jnp.float32),
            pltpu.VMEM((_B_PER_W,), jnp.int32),
            pltpu.VMEM((N_GRID * EMBED_DIM,), jnp.float32),
            pltpu.VMEM((2, _CHUNK, EMBED_DIM), jnp.float32),
            pltpu.SemaphoreType.DMA,
            pltpu.SemaphoreType.DMA,
            pltpu.SemaphoreType.DMA,
            pltpu.SemaphoreType.DMA,
        ],
        compiler_params=pltpu.CompilerParams(needs_layout_passes=False),
    )
    def sc_fused(table_hbm, label_hbm, ts_hbm, tgrid_hbm, out_hbm,
                 lbl_v, t_v, q_v, tg_v, rows2, g0, g1, w0, w1):
        sid = lax.axis_index("s")
        wid = sid * _NC + lax.axis_index("c")
        base = wid * _B_PER_W

        pltpu.sync_copy(tgrid_hbm, tg_v)
        pltpu.sync_copy(label_hbm.at[pl.ds(base, _B_PER_W)], lbl_v)
        pltpu.sync_copy(ts_hbm.at[pl.ds(base, _B_PER_W)], t_v)
        for j in range(_B_PER_W // 16):
            t16 = t_v[pl.ds(j * 16, 16)]
            q_v[pl.ds(j * 16, 16)] = (
                t16 * float(N_GRID - 1) + 0.5).astype(jnp.int32)

        gsem = (g0, g1)
        wsem = (w0, w1)
        gdesc = [None, None]
        wdesc = [None, None]
        lane = lax.iota(jnp.int32, 16)

        def issue_gather(c):
            b = c & 1
            gdesc[b] = pltpu.async_copy(
                table_hbm.at[lbl_v.at[pl.ds(c * _CHUNK, _CHUNK)]],
                rows2.at[b], gsem[b])

        def finish_chunk(c):
            b = c & 1
            gdesc[b].wait()

            def add_body(r, _):
                qb = plsc.load_gather(
                    q_v, [jnp.full((16,), c * _CHUNK + r, jnp.int32)])
                idx0 = qb * EMBED_DIM + lane
                for k in range(EMBED_DIM // 16):
                    sl = pl.ds(k * 16, 16)
                    tv = plsc.load_gather(tg_v, [idx0 + (k * 16)])
                    rows2[b, r, sl] = rows2[b, r, sl] + tv
                return ()

            lax.fori_loop(0, _CHUNK, add_body, (), unroll=False)
            wdesc[b] = pltpu.async_copy(
                rows2.at[b], out_hbm.at[pl.ds(base + c * _CHUNK, _CHUNK)],
                wsem[b])

        issue_gather(0)
        for c in range(1, _N_CHUNKS):
            b = c & 1
            if c >= 2:
                wdesc[b].wait()
            issue_gather(c)
            finish_chunk(c - 1)
        finish_chunk(_N_CHUNKS - 1)
        wdesc[0].wait()
        wdesc[1].wait()

    return sc_fused


def kernel(label, timestep, emb_table, W1, b1, W2, b2):
    t_grid = _tc_grid_mlp(
        jnp.asarray(_FREQ),
        W1,
        b1.reshape(1, -1),
        W2,
        b2.reshape(1, -1),
    )
    return _make_sc_fused()(emb_table, label.astype(jnp.int32), timestep,
                            t_grid.reshape(-1))


# speedup vs baseline: 1.5899x; 1.5899x over previous
import functools

import jax
import jax.numpy as jnp
import numpy as np
from jax import lax
from jax.experimental import pallas as pl
from jax.experimental.pallas import tpu as pltpu
from jax.experimental.pallas import tpu_sc as plsc

NUM_CLASSES = 1000
EMBED_DIM = 512
INTER_DIM = 2048
BATCH = 16384
HALF_DIM = EMBED_DIM // 2
N_GRID = 512


_FREQ = np.exp(
    -np.log(10000.0) * np.arange(HALF_DIM, dtype=np.float32) / HALF_DIM
).astype(np.float32).reshape(1, HALF_DIM)


def _grid_mlp_body(freq_ref, w1_ref, b1_ref, w2_ref, b2_ref, t_ref):
    g = lax.broadcasted_iota(jnp.int32, (N_GRID, HALF_DIM), 0).astype(jnp.float32)
    x = (g * (1.0 / (N_GRID - 1))) * freq_ref[0, :][None, :]
    y = x * x
    s = x * (1.0 + y * (-1.0 / 6.0 + y * (1.0 / 120.0 + y * (-1.0 / 5040.0))))
    c = 1.0 + y * (-0.5 + y * (1.0 / 24.0 + y * (-1.0 / 720.0)))
    emb = jnp.concatenate([s, c], axis=-1).astype(jnp.bfloat16)
    h = lax.dot_general(
        emb, w1_ref[...].astype(jnp.bfloat16),
        (((1,), (1,)), ((), ())),
        preferred_element_type=jnp.float32,
    ) + b1_ref[0, :][None, :]
    h = h * jax.nn.sigmoid(h)
    h = h.astype(jnp.bfloat16)
    out = lax.dot_general(
        h, w2_ref[...].astype(jnp.bfloat16),
        (((1,), (1,)), ((), ())),
        preferred_element_type=jnp.float32,
    ) + b2_ref[0, :][None, :]
    t_ref[...] = out


def _tc_grid_mlp(freq, w1, b1, w2, b2):
    return pl.pallas_call(
        _grid_mlp_body,
        out_shape=jax.ShapeDtypeStruct((N_GRID, EMBED_DIM), jnp.float32),
    )(freq, w1, b1, w2, b2)


_NC = 2
_NS = 16
_NW = _NC * _NS
_B_PER_W = BATCH // _NW
_CHUNK = 32
_N_CHUNKS = _B_PER_W // _CHUNK


@functools.cache
def _make_sc_fused():
    mesh = plsc.VectorSubcoreMesh(core_axis_name="c", subcore_axis_name="s")

    @functools.partial(
        pl.kernel,
        mesh=mesh,
        out_type=jax.ShapeDtypeStruct((BATCH, EMBED_DIM), jnp.float32),
        scratch_types=[
            pltpu.VMEM((_N_CHUNKS, _CHUNK), jnp.int32),
            pltpu.VMEM((_N_CHUNKS, _CHUNK), jnp.float32),
            pltpu.VMEM((_N_CHUNKS, _CHUNK), jnp.int32),
            pltpu.VMEM((2, _CHUNK, EMBED_DIM), jnp.float32),
            pltpu.VMEM((2, _CHUNK, EMBED_DIM), jnp.float32),
            pltpu.SemaphoreType.DMA,
            pltpu.SemaphoreType.DMA,
            pltpu.SemaphoreType.DMA,
            pltpu.SemaphoreType.DMA,
        ],
    )
    def sc_fused(table_hbm, label_hbm, ts_hbm, tgrid_hbm, out_hbm,
                 lbl_v, t_v, q_v, rows2, trow2, g0, g1, w0, w1):
        sid = lax.axis_index("s")
        wid = sid * _NC + lax.axis_index("c")
        base = wid * _B_PER_W

        for c in range(_N_CHUNKS):
            pltpu.sync_copy(label_hbm.at[pl.ds(base + c * _CHUNK, _CHUNK)],
                            lbl_v.at[c])
            pltpu.sync_copy(ts_hbm.at[pl.ds(base + c * _CHUNK, _CHUNK)],
                            t_v.at[c])
        for c in range(_N_CHUNKS):
            for j in range(_CHUNK // 16):
                t16 = t_v[c, pl.ds(j * 16, 16)]
                q_v[c, pl.ds(j * 16, 16)] = (
                    t16 * float(N_GRID - 1) + 0.5).astype(jnp.int32)

        gsem = (g0, g1)
        wsem = (w0, w1)
        gdesc = [None, None]
        wdesc = [None, None]

        def issue_gathers(c):
            b = c & 1
            ga = pltpu.async_copy(table_hbm.at[lbl_v.at[c]], rows2.at[b],
                                  gsem[b])
            gb = pltpu.async_copy(tgrid_hbm.at[q_v.at[c]], trow2.at[b],
                                  gsem[b])
            gdesc[b] = (ga, gb)

        def finish_chunk(c):
            b = c & 1
            ga, gb = gdesc[b]
            ga.wait()
            gb.wait()

            def add_body(r, _):
                for k in range(EMBED_DIM // 16):
                    sl = pl.ds(k * 16, 16)
                    rows2[b, r, sl] = rows2[b, r, sl] + trow2[b, r, sl]
                return ()

            lax.fori_loop(0, _CHUNK, add_body, (), unroll=False)
            wdesc[b] = pltpu.async_copy(
                rows2.at[b], out_hbm.at[pl.ds(base + c * _CHUNK, _CHUNK)],
                wsem[b])

        issue_gathers(0)
        for c in range(1, _N_CHUNKS):
            b = c & 1
            if c >= 2:
                wdesc[b].wait()
            issue_gathers(c)
            finish_chunk(c - 1)
        finish_chunk(_N_CHUNKS - 1)
        wdesc[0].wait()
        wdesc[1].wait()

    return sc_fused


def kernel(label, timestep, emb_table, W1, b1, W2, b2):
    t_grid = _tc_grid_mlp(
        jnp.asarray(_FREQ),
        W1,
        b1.reshape(1, -1),
        W2,
        b2.reshape(1, -1),
    )
    return _make_sc_fused()(emb_table, label.astype(jnp.int32), timestep,
                            t_grid)


# speedup vs baseline: 1.7447x; 1.0973x over previous
import functools

import jax
import jax.numpy as jnp
import numpy as np
from jax import lax
from jax.experimental import pallas as pl
from jax.experimental.pallas import tpu as pltpu
from jax.experimental.pallas import tpu_sc as plsc

NUM_CLASSES = 1000
EMBED_DIM = 512
INTER_DIM = 2048
BATCH = 16384
HALF_DIM = EMBED_DIM // 2
N_GRID = 512


_FREQ = np.exp(
    -np.log(10000.0) * np.arange(HALF_DIM, dtype=np.float32) / HALF_DIM
).astype(np.float32).reshape(1, HALF_DIM)


def _grid_mlp_body(freq_ref, w1_ref, b1_ref, w2_ref, b2_ref, t_ref):
    g = lax.broadcasted_iota(jnp.int32, (N_GRID, HALF_DIM), 0).astype(jnp.float32)
    x = (g * (1.0 / (N_GRID - 1))) * freq_ref[0, :][None, :]
    y = x * x
    s = x * (1.0 + y * (-1.0 / 6.0 + y * (1.0 / 120.0 + y * (-1.0 / 5040.0))))
    c = 1.0 + y * (-0.5 + y * (1.0 / 24.0 + y * (-1.0 / 720.0)))
    emb = jnp.concatenate([s, c], axis=-1).astype(jnp.bfloat16)
    h = lax.dot_general(
        emb, w1_ref[...].astype(jnp.bfloat16),
        (((1,), (1,)), ((), ())),
        preferred_element_type=jnp.float32,
    ) + b1_ref[0, :][None, :]
    h = h * jax.nn.sigmoid(h)
    h = h.astype(jnp.bfloat16)
    out = lax.dot_general(
        h, w2_ref[...].astype(jnp.bfloat16),
        (((1,), (1,)), ((), ())),
        preferred_element_type=jnp.float32,
    ) + b2_ref[0, :][None, :]
    t_ref[...] = out


def _tc_grid_mlp(freq, w1, b1, w2, b2):
    return pl.pallas_call(
        _grid_mlp_body,
        out_shape=jax.ShapeDtypeStruct((N_GRID, EMBED_DIM), jnp.float32),
    )(freq, w1, b1, w2, b2)


_NC = 2
_NS = 16
_NW = _NC * _NS
_B_PER_W = BATCH // _NW
_CHUNK = 64
_N_CHUNKS = _B_PER_W // _CHUNK


@functools.cache
def _make_sc_fused():
    mesh = plsc.VectorSubcoreMesh(core_axis_name="c", subcore_axis_name="s")

    @functools.partial(
        pl.kernel,
        mesh=mesh,
        out_type=jax.ShapeDtypeStruct((BATCH, EMBED_DIM), jnp.float32),
        scratch_types=[
            pltpu.VMEM((_N_CHUNKS, _CHUNK), jnp.int32),
            pltpu.VMEM((_N_CHUNKS, _CHUNK), jnp.float32),
            pltpu.VMEM((_N_CHUNKS, _CHUNK), jnp.int32),
            pltpu.VMEM((2, _CHUNK, EMBED_DIM), jnp.float32),
            pltpu.VMEM((_CHUNK, EMBED_DIM), jnp.float32),
            pltpu.SemaphoreType.DMA,
            pltpu.SemaphoreType.DMA,
            pltpu.SemaphoreType.DMA,
            pltpu.SemaphoreType.DMA,
            pltpu.SemaphoreType.DMA,
        ],
    )
    def sc_fused(table_hbm, label_hbm, ts_hbm, tgrid_hbm, out_hbm,
                 lbl_v, t_v, q_v, rows2, trow2, g0, g1, w0, w1, tsem):
        sid = lax.axis_index("s")
        wid = sid * _NC + lax.axis_index("c")
        base = wid * _B_PER_W

        for c in range(_N_CHUNKS):
            pltpu.sync_copy(label_hbm.at[pl.ds(base + c * _CHUNK, _CHUNK)],
                            lbl_v.at[c])
            pltpu.sync_copy(ts_hbm.at[pl.ds(base + c * _CHUNK, _CHUNK)],
                            t_v.at[c])
        for c in range(_N_CHUNKS):
            for j in range(_CHUNK // 16):
                t16 = t_v[c, pl.ds(j * 16, 16)]
                q_v[c, pl.ds(j * 16, 16)] = (
                    t16 * float(N_GRID - 1) + 0.5).astype(jnp.int32)

        gsem = (g0, g1)
        wsem = (w0, w1)
        gdesc = [None, None]
        tdesc = [None]
        wdesc = [None, None]

        def issue_rows_gather(c):
            b = c & 1
            gdesc[b] = pltpu.async_copy(table_hbm.at[lbl_v.at[c]],
                                        rows2.at[b], gsem[b])

        def issue_trow_gather(c):
            tdesc[0] = pltpu.async_copy(tgrid_hbm.at[q_v.at[c]], trow2, tsem)

        issue_rows_gather(0)
        issue_trow_gather(0)
        for c in range(_N_CHUNKS):
            b = c & 1
            if c + 1 < _N_CHUNKS:
                if c >= 1:
                    wdesc[(c + 1) & 1].wait()
                issue_rows_gather(c + 1)
            gdesc[b].wait()
            tdesc[0].wait()

            def add_body(r, _):
                for k in range(EMBED_DIM // 16):
                    sl = pl.ds(k * 16, 16)
                    rows2[b, r, sl] = rows2[b, r, sl] + trow2[r, sl]
                return ()

            lax.fori_loop(0, _CHUNK, add_body, (), unroll=False)
            wdesc[b] = pltpu.async_copy(
                rows2.at[b], out_hbm.at[pl.ds(base + c * _CHUNK, _CHUNK)],
                wsem[b])
            if c + 1 < _N_CHUNKS:
                issue_trow_gather(c + 1)
        wdesc[0].wait()
        wdesc[1].wait()

    return sc_fused


def kernel(label, timestep, emb_table, W1, b1, W2, b2):
    t_grid = _tc_grid_mlp(
        jnp.asarray(_FREQ),
        W1,
        b1.reshape(1, -1),
        W2,
        b2.reshape(1, -1),
    )
    return _make_sc_fused()(emb_table, label.astype(jnp.int32), timestep,
                            t_grid)


# speedup vs baseline: 1.7475x; 1.0016x over previous
import functools

import jax
import jax.numpy as jnp
import numpy as np
from jax import lax
from jax.experimental import pallas as pl
from jax.experimental.pallas import tpu as pltpu
from jax.experimental.pallas import tpu_sc as plsc

NUM_CLASSES = 1000
EMBED_DIM = 512
INTER_DIM = 2048
BATCH = 16384
HALF_DIM = EMBED_DIM // 2
N_GRID = 512


_FREQ = np.exp(
    -np.log(10000.0) * np.arange(HALF_DIM, dtype=np.float32) / HALF_DIM
).astype(np.float32).reshape(1, HALF_DIM)


def _grid_mlp_body(freq_ref, w1_ref, b1_ref, w2_ref, b2_ref, t_ref):
    g = lax.broadcasted_iota(jnp.int32, (N_GRID, HALF_DIM), 0).astype(jnp.float32)
    x = (g * (1.0 / (N_GRID - 1))) * freq_ref[0, :][None, :]
    y = x * x
    s = x * (1.0 + y * (-1.0 / 6.0 + y * (1.0 / 120.0 + y * (-1.0 / 5040.0))))
    c = 1.0 + y * (-0.5 + y * (1.0 / 24.0 + y * (-1.0 / 720.0)))
    emb = jnp.concatenate([s, c], axis=-1).astype(jnp.bfloat16)
    h = lax.dot_general(
        emb, w1_ref[...].astype(jnp.bfloat16),
        (((1,), (1,)), ((), ())),
        preferred_element_type=jnp.float32,
    ) + b1_ref[0, :][None, :]
    h = h * jax.nn.sigmoid(h)
    h = h.astype(jnp.bfloat16)
    out = lax.dot_general(
        h, w2_ref[...].astype(jnp.bfloat16),
        (((1,), (1,)), ((), ())),
        preferred_element_type=jnp.float32,
    ) + b2_ref[0, :][None, :]
    t_ref[...] = out


def _tc_grid_mlp(freq, w1, b1, w2, b2):
    return pl.pallas_call(
        _grid_mlp_body,
        out_shape=jax.ShapeDtypeStruct((N_GRID, EMBED_DIM), jnp.float32),
    )(freq, w1, b1, w2, b2)


_NC = 2
_NS = 16
_NW = _NC * _NS
_B_PER_W = BATCH // _NW
_CHUNK = 64
_N_CHUNKS = _B_PER_W // _CHUNK


@functools.cache
def _make_sc_fused():
    mesh = plsc.VectorSubcoreMesh(core_axis_name="c", subcore_axis_name="s")

    @functools.partial(
        pl.kernel,
        mesh=mesh,
        out_type=jax.ShapeDtypeStruct((BATCH, EMBED_DIM), jnp.float32),
        scratch_types=[
            pltpu.VMEM((_N_CHUNKS, _CHUNK), jnp.int32),
            pltpu.VMEM((_N_CHUNKS, _CHUNK), jnp.float32),
            pltpu.VMEM((_N_CHUNKS * 2, _CHUNK // 2), jnp.int32),
            pltpu.VMEM((2, _CHUNK, EMBED_DIM), jnp.float32),
            pltpu.VMEM((2, _CHUNK // 2, EMBED_DIM), jnp.float32),
            pltpu.SemaphoreType.DMA,
            pltpu.SemaphoreType.DMA,
            pltpu.SemaphoreType.DMA,
            pltpu.SemaphoreType.DMA,
            pltpu.SemaphoreType.DMA,
            pltpu.SemaphoreType.DMA,
        ],
    )
    def sc_fused(table_hbm, label_hbm, ts_hbm, tgrid_hbm, out_hbm,
                 lbl_v, t_v, q_v, rows2, trow2, g0, g1, w0, w1, t0, t1):
        sid = lax.axis_index("s")
        wid = sid * _NC + lax.axis_index("c")
        base = wid * _B_PER_W

        for c in range(_N_CHUNKS):
            pltpu.sync_copy(label_hbm.at[pl.ds(base + c * _CHUNK, _CHUNK)],
                            lbl_v.at[c])
            pltpu.sync_copy(ts_hbm.at[pl.ds(base + c * _CHUNK, _CHUNK)],
                            t_v.at[c])
        for c in range(_N_CHUNKS):
            for j in range(_CHUNK // 16):
                t16 = t_v[c, pl.ds(j * 16, 16)]
                s = 2 * c + j // 2
                q_v[s, pl.ds((j % 2) * 16, 16)] = (
                    t16 * float(N_GRID - 1) + 0.5).astype(jnp.int32)

        gsem = (g0, g1)
        wsem = (w0, w1)
        tsems = (t0, t1)
        gdesc = [None, None]
        tdesc = [None, None]
        wdesc = [None, None]
        n_sub = _N_CHUNKS * 2
        half = _CHUNK // 2

        def issue_rows_gather(c):
            b = c & 1
            gdesc[b] = pltpu.async_copy(table_hbm.at[lbl_v.at[c]],
                                        rows2.at[b], gsem[b])

        def issue_trow_gather(s):
            sb = s & 1
            tdesc[sb] = pltpu.async_copy(tgrid_hbm.at[q_v.at[s]],
                                         trow2.at[sb], tsems[sb])

        issue_rows_gather(0)
        issue_trow_gather(0)
        issue_trow_gather(1)
        for c in range(_N_CHUNKS):
            b = c & 1
            if c + 1 < _N_CHUNKS:
                if c >= 1:
                    wdesc[(c + 1) & 1].wait()
                issue_rows_gather(c + 1)
            gdesc[b].wait()
            for h in range(2):
                s = 2 * c + h
                sb = s & 1
                tdesc[sb].wait()

                def add_body(r, _):
                    for k in range(EMBED_DIM // 16):
                        sl = pl.ds(k * 16, 16)
                        rr = h * half + r
                        rows2[b, rr, sl] = rows2[b, rr, sl] + trow2[sb, r, sl]
                    return ()

                lax.fori_loop(0, half, add_body, (), unroll=False)
                if s + 2 < n_sub:
                    issue_trow_gather(s + 2)
            wdesc[b] = pltpu.async_copy(
                rows2.at[b], out_hbm.at[pl.ds(base + c * _CHUNK, _CHUNK)],
                wsem[b])
        wdesc[0].wait()
        wdesc[1].wait()

    return sc_fused


def kernel(label, timestep, emb_table, W1, b1, W2, b2):
    t_grid = _tc_grid_mlp(
        jnp.asarray(_FREQ),
        W1,
        b1.reshape(1, -1),
        W2,
        b2.reshape(1, -1),
    )
    return _make_sc_fused()(emb_table, label.astype(jnp.int32), timestep,
                            t_grid)


# speedup vs baseline: 1.8676x; 1.0687x over previous
import functools

import jax
import jax.numpy as jnp
import numpy as np
from jax import lax
from jax.experimental import pallas as pl
from jax.experimental.pallas import tpu as pltpu
from jax.experimental.pallas import tpu_sc as plsc

NUM_CLASSES = 1000
EMBED_DIM = 512
INTER_DIM = 2048
BATCH = 16384
HALF_DIM = EMBED_DIM // 2
N_GRID = 512


_FREQ = np.exp(
    -np.log(10000.0) * np.arange(HALF_DIM, dtype=np.float32) / HALF_DIM
).astype(np.float32).reshape(1, HALF_DIM)


def _grid_mlp_body(freq_ref, w1_ref, b1_ref, w2_ref, b2_ref, t_ref):
    g = lax.broadcasted_iota(jnp.int32, (N_GRID, HALF_DIM), 0).astype(jnp.float32)
    x = (g * (1.0 / (N_GRID - 1))) * freq_ref[0, :][None, :]
    y = x * x
    s = x * (1.0 + y * (-1.0 / 6.0 + y * (1.0 / 120.0 + y * (-1.0 / 5040.0))))
    c = 1.0 + y * (-0.5 + y * (1.0 / 24.0 + y * (-1.0 / 720.0)))
    emb = jnp.concatenate([s, c], axis=-1).astype(jnp.bfloat16)
    h = lax.dot_general(
        emb, w1_ref[...].astype(jnp.bfloat16),
        (((1,), (1,)), ((), ())),
        preferred_element_type=jnp.float32,
    ) + b1_ref[0, :][None, :]
    h = h * jax.nn.sigmoid(h)
    h = h.astype(jnp.bfloat16)
    out = lax.dot_general(
        h, w2_ref[...].astype(jnp.bfloat16),
        (((1,), (1,)), ((), ())),
        preferred_element_type=jnp.float32,
    ) + b2_ref[0, :][None, :]
    t_ref[...] = out


def _tc_grid_mlp(freq, w1, b1, w2, b2):
    return pl.pallas_call(
        _grid_mlp_body,
        out_shape=jax.ShapeDtypeStruct((N_GRID, EMBED_DIM), jnp.float32),
    )(freq, w1, b1, w2, b2)


_NC = 2
_NS = 16
_NW = _NC * _NS
_B_PER_W = BATCH // _NW
_CHUNK = 64
_N_CHUNKS = _B_PER_W // _CHUNK


@functools.cache
def _make_sc_fused():
    mesh = plsc.VectorSubcoreMesh(core_axis_name="c", subcore_axis_name="s")

    @functools.partial(
        pl.kernel,
        mesh=mesh,
        out_type=jax.ShapeDtypeStruct((BATCH, EMBED_DIM), jnp.float32),
        scratch_types=[
            pltpu.VMEM((_B_PER_W,), jnp.int32),
            pltpu.VMEM((_B_PER_W,), jnp.float32),
            pltpu.VMEM((_B_PER_W,), jnp.int32),
            pltpu.VMEM((2, _CHUNK, EMBED_DIM), jnp.float32),
            pltpu.VMEM((2, _CHUNK // 2, EMBED_DIM), jnp.float32),
            pltpu.SemaphoreType.DMA,
            pltpu.SemaphoreType.DMA,
            pltpu.SemaphoreType.DMA,
            pltpu.SemaphoreType.DMA,
            pltpu.SemaphoreType.DMA,
            pltpu.SemaphoreType.DMA,
        ],
    )
    def sc_fused(table_hbm, label_hbm, ts_hbm, tgrid_hbm, out_hbm,
                 lbl_v, t_v, q_v, rows2, trow2, g0, g1, w0, w1, t0, t1):
        sid = lax.axis_index("s")
        wid = sid * _NC + lax.axis_index("c")
        base = wid * _B_PER_W

        pltpu.sync_copy(label_hbm.at[pl.ds(base, _B_PER_W)], lbl_v)
        pltpu.sync_copy(ts_hbm.at[pl.ds(base, _B_PER_W)], t_v)

        gsem = (g0, g1)
        wsem = (w0, w1)
        tsems = (t0, t1)
        gdesc = [None, None]
        tdesc = [None, None]
        wdesc = [None, None]
        n_sub = _N_CHUNKS * 2
        half = _CHUNK // 2

        def issue_rows_gather(c):
            b = c & 1
            gdesc[b] = pltpu.async_copy(
                table_hbm.at[lbl_v.at[pl.ds(c * _CHUNK, _CHUNK)]],
                rows2.at[b], gsem[b])

        def issue_trow_gather(s):
            sb = s & 1
            tdesc[sb] = pltpu.async_copy(
                tgrid_hbm.at[q_v.at[pl.ds(s * half, half)]],
                trow2.at[sb], tsems[sb])

        issue_rows_gather(0)
        for j in range(_B_PER_W // 16):
            t16 = t_v[pl.ds(j * 16, 16)]
            q_v[pl.ds(j * 16, 16)] = (
                t16 * float(N_GRID - 1) + 0.5).astype(jnp.int32)
        issue_trow_gather(0)
        issue_trow_gather(1)
        for c in range(_N_CHUNKS):
            b = c & 1
            if c + 1 < _N_CHUNKS:
                if c >= 1:
                    wdesc[(c + 1) & 1].wait()
                issue_rows_gather(c + 1)
            gdesc[b].wait()
            for h in range(2):
                s = 2 * c + h
                sb = s & 1
                tdesc[sb].wait()

                def add_body(r, _):
                    for k in range(EMBED_DIM // 16):
                        sl = pl.ds(k * 16, 16)
                        rr = h * half + r
                        rows2[b, rr, sl] = rows2[b, rr, sl] + trow2[sb, r, sl]
                    return ()

                lax.fori_loop(0, half, add_body, (), unroll=False)
                if s + 2 < n_sub:
                    issue_trow_gather(s + 2)
            wdesc[b] = pltpu.async_copy(
                rows2.at[b], out_hbm.at[pl.ds(base + c * _CHUNK, _CHUNK)],
                wsem[b])
        wdesc[0].wait()
        wdesc[1].wait()

    return sc_fused


def kernel(label, timestep, emb_table, W1, b1, W2, b2):
    t_grid = _tc_grid_mlp(
        jnp.asarray(_FREQ),
        W1,
        b1.reshape(1, -1),
        W2,
        b2.reshape(1, -1),
    )
    return _make_sc_fused()(emb_table, label.astype(jnp.int32), timestep,
                            t_grid)
